# Initial kernel scaffold; baseline (speedup 1.0000x reference)
import jax
import jax.numpy as jnp
from jax.experimental import pallas as pl


def kernel(x, W_emb, W_att, b_att):
    raise NotImplementedError("write your pallas kernel here")


# speedup vs baseline: 2.1298x; 2.1298x over previous
import functools

import jax
import jax.numpy as jnp
from jax import lax
from jax.experimental import pallas as pl
from jax.experimental.pallas import tpu as pltpu
from jax.experimental.pallas import tpu_sc as plsc

VOCAB = 100000
DIM = 128
BATCH = 4096
SEQ = 200
LPAD = 208
HALF = LPAD // 2
NWORK = 32
RPT = BATCH // NWORK
VBLK = 5000
NCHUNK = LPAD // 16


def _score_body(w_ref, a_ref, b_ref, o_ref):
    z = jnp.dot(w_ref[...], a_ref[...], preferred_element_type=jnp.float32)
    o_ref[...] = jnp.tanh(z + b_ref[0])


def _scores(W_emb, W_att, b_att):
    return pl.pallas_call(
        _score_body,
        grid=(VOCAB // VBLK,),
        in_specs=[
            pl.BlockSpec((VBLK, DIM), lambda i: (i, 0)),
            pl.BlockSpec((DIM, 1), lambda i: (0, 0)),
            pl.BlockSpec(memory_space=pltpu.SMEM),
        ],
        out_specs=pl.BlockSpec((VBLK, 1), lambda i: (i, 0)),
        out_shape=jax.ShapeDtypeStruct((VOCAB, 1), jnp.float32),
    )(W_emb, W_att, b_att)


def _sc_body(x2_hbm, s_hbm, emb_hbm, out_hbm,
             idx_v, rows_v, sc_v, w_v, out_v, sem):
    wid = lax.axis_index("s") * 2 + lax.axis_index("c")
    base = wid * RPT
    pltpu.sync_copy(x2_hbm.at[pl.ds(base * 2, 2 * RPT)], idx_v)

    def row_body(i, carry):
        c0 = pltpu.async_copy(s_hbm.at[idx_v.at[2 * i]],
                              sc_v.at[pl.ds(0, HALF)], sem)
        c1 = pltpu.async_copy(s_hbm.at[idx_v.at[2 * i + 1]],
                              sc_v.at[pl.ds(HALF, HALF)], sem)
        c2 = pltpu.async_copy(emb_hbm.at[idx_v.at[2 * i]],
                              rows_v.at[pl.ds(0, HALF)], sem)
        c3 = pltpu.async_copy(emb_hbm.at[idx_v.at[2 * i + 1]],
                              rows_v.at[pl.ds(HALF, HALF)], sem)
        c0.wait(); c1.wait(); c2.wait(); c3.wait()

        def sum_body(c, acc):
            t = sc_v[pl.ds(c * 16, 16)]
            pos = c * 16 + lax.iota(jnp.int32, 16)
            e = jnp.where(pos < SEQ, jnp.exp(t), 0.0)
            w_v[pl.ds(c * 16, 16)] = e
            return acc + e

        acc = lax.fori_loop(0, NCHUNK, sum_body, jnp.zeros((16,), jnp.float32))
        total = acc[0]
        for j in range(1, 16):
            total = total + acc[j]
        inv = 1.0 / jnp.full((16,), total, jnp.float32)

        def chunk_body(c, accs):
            w16 = w_v[pl.ds(c * 16, 16)] * inv
            accs = list(accs)
            for j in range(16):
                w = w16[j]
                for k in range(8):
                    accs[k] = accs[k] + w * rows_v[c * 16 + j, pl.ds(k * 16, 16)]
            return tuple(accs)

        accs = lax.fori_loop(
            0, NCHUNK, chunk_body,
            tuple(jnp.zeros((16,), jnp.float32) for _ in range(8)))
        for c in range(8):
            out_v[i, pl.ds(c * 16, 16)] = accs[c]
        return carry

    lax.fori_loop(0, RPT, row_body, 0)
    pltpu.sync_copy(out_v, out_hbm.at[pl.ds(base, RPT)])


_sc_call = functools.partial(
    pl.kernel,
    out_type=jax.ShapeDtypeStruct((BATCH, DIM), jnp.float32),
    mesh=plsc.VectorSubcoreMesh(core_axis_name="c", subcore_axis_name="s"),
    scratch_types=[
        pltpu.VMEM((2 * RPT, HALF), jnp.int32),
        pltpu.VMEM((LPAD, DIM), jnp.float32),
        pltpu.VMEM((LPAD,), jnp.float32),
        pltpu.VMEM((LPAD,), jnp.float32),
        pltpu.VMEM((RPT, DIM), jnp.float32),
        pltpu.SemaphoreType.DMA,
    ],
)


@jax.jit
def kernel(x, W_emb, W_att, b_att):
    s = _scores(W_emb, W_att, b_att).reshape(VOCAB)
    x = x.astype(jnp.int32)
    x2 = jnp.concatenate(
        [x, jnp.zeros((BATCH, LPAD - SEQ), jnp.int32)], axis=1
    ).reshape(2 * BATCH, HALF)
    return _sc_call(_sc_body)(x2, s, W_emb)
